# baseline (speedup 1.0000x reference)
import functools

import jax
import jax.numpy as jnp
from jax import lax
from jax.experimental import pallas as pl
from jax.experimental.pallas import tpu as pltpu
from jax.experimental.pallas import tpu_sc as plsc

NC, NS = 2, 16
NW = NC * NS
NB = 4
PD = 128
PH = 64
L = 16


@functools.lru_cache(maxsize=None)
def _build(BATCH, HIST, D):
    b_per_w = BATCH // NW
    n_chunks = b_per_w // NB
    assert n_chunks % 2 == 0 and n_chunks >= 6

    mesh = plsc.VectorSubcoreMesh(
        core_axis_name="c", subcore_axis_name="s",
        num_cores=NC, num_subcores=NS)

    @functools.partial(
        pl.kernel,
        mesh=mesh,
        compiler_params=pltpu.CompilerParams(use_tc_tiling_on_sc=True),
        out_type=jax.ShapeDtypeStruct((BATCH, HIST, D), jnp.float32),
        scratch_types=[
            pltpu.VMEM((2 * NB, PH), jnp.int32),
            pltpu.VMEM((2 * NB, PH), jnp.int32),
            pltpu.VMEM((2 * NB, PH, PD), jnp.float32),
            pltpu.VMEM((2 * NB, HIST, D), jnp.float32),
            pltpu.SemaphoreType.DMA((2,)),
            pltpu.SemaphoreType.DMA((2,)),
        ],
    )
    def gather_kernel(idx_hbm, table_hbm, out_hbm, idx_v, par_v, rows_v,
                      comp_v, gsem, osem):
        wid = lax.axis_index("s") * NC + lax.axis_index("c")
        b_base = wid * b_per_w

        def fire(c, s):
            b0 = b_base + c * NB
            pltpu.sync_copy(idx_hbm.at[pl.ds(b0, NB)],
                            idx_v.at[pl.ds(s * NB, NB)])
            for i in range(NB):
                for g in range(PH // L):
                    tid = idx_v[s * NB + i, pl.ds(g * L, L)]
                    idx_v[s * NB + i, pl.ds(g * L, L)] = tid >> 1
                    par_v[s * NB + i, pl.ds(g * L, L)] = tid & 1
            for i in range(NB):
                pltpu.async_copy(
                    table_hbm.at[idx_v.at[s * NB + i]],
                    rows_v.at[s * NB + i],
                    gsem.at[s])

        def wait_gathers(s):
            for i in range(NB):
                pltpu.make_async_copy(
                    table_hbm.at[pl.ds(0, PH)], rows_v.at[s * NB + i],
                    gsem.at[s]).wait()

        def compact(s):
            def comp_rows(i, base, ks):
                pv = par_v[s * NB + i, pl.ds(base, L)]
                for k in ks:
                    h = base + k
                    for j in range(D // L):
                        comp_v[s * NB + i, h, pl.ds(j * L, L)] = (
                            rows_v[s * NB + i, h, pl.ds(j * L, L)])

                    @pl.when(pv[k] != 0)
                    def _():
                        for j in range(D // L):
                            comp_v[s * NB + i, h, pl.ds(j * L, L)] = (
                                rows_v[s * NB + i, h, pl.ds(D + j * L, L)])

            def comp_g(g, carry):
                for i in range(NB):
                    comp_rows(i, g * L, range(L))
                return carry

            lax.fori_loop(0, HIST // L, comp_g, 0)
            rem = HIST % L
            if rem:
                for i in range(NB):
                    comp_rows(i, (HIST // L) * L, range(rem))

        def write(c, s):
            b0 = b_base + c * NB
            pltpu.async_copy(
                comp_v.at[pl.ds(s * NB, NB)], out_hbm.at[pl.ds(b0, NB)],
                osem.at[s])

        def drain_out(s):
            pltpu.make_async_copy(
                out_hbm.at[pl.ds(b_base, NB)], comp_v.at[pl.ds(s * NB, NB)],
                osem.at[s]).wait()

        def retire(c, s):
            wait_gathers(s)
            compact(s)
            write(c, s)

        fire(0, 0)
        fire(1, 1)
        retire(0, 0)
        drain_out(0)
        fire(2, 0)
        retire(1, 1)

        def body(g, carry):
            c0 = 2 * g
            drain_out(1)
            fire(c0 + 1, 1)
            retire(c0, 0)
            drain_out(0)
            fire(c0 + 2, 0)
            retire(c0 + 1, 1)
            return carry

        lax.fori_loop(1, n_chunks // 2 - 1, body, 0)

        c0 = n_chunks - 2
        drain_out(1)
        fire(c0 + 1, 1)
        retire(c0, 0)
        retire(c0 + 1, 1)
        drain_out(0)
        drain_out(1)

    return gather_kernel


def kernel(token_ids, w):
    BATCH, HIST = token_ids.shape
    V, D = w.shape
    idx2 = jnp.pad(token_ids.astype(jnp.int32), ((0, 0), (0, PH - HIST)))
    w2 = w.reshape(V // 2, 2 * D)
    return _build(BATCH, HIST, D)(idx2, w2)


# speedup vs baseline: 8.2414x; 8.2414x over previous
import functools

import jax
import jax.numpy as jnp
from jax import lax
from jax.experimental import pallas as pl
from jax.experimental.pallas import tpu as pltpu
from jax.experimental.pallas import tpu_sc as plsc

NC, NS = 2, 16
NW = NC * NS
IW = 128
CPW_IR = 4
RPC = CPW_IR * IW


@functools.lru_cache(maxsize=None)
def _build(B, D):
    n_ir = B // IW
    ir_per_w = n_ir // NW
    n_chunks = ir_per_w // CPW_IR
    assert n_chunks % 2 == 0 and n_chunks >= 6

    mesh = plsc.VectorSubcoreMesh(
        core_axis_name="c", subcore_axis_name="s",
        num_cores=NC, num_subcores=NS)

    @functools.partial(
        pl.kernel,
        mesh=mesh,
        compiler_params=pltpu.CompilerParams(use_tc_tiling_on_sc=False),
        out_type=jax.ShapeDtypeStruct((B, D), jnp.float32),
        scratch_types=[
            pltpu.VMEM((2, CPW_IR, IW), jnp.int32),
            pltpu.VMEM((2, RPC, D), jnp.float32),
            pltpu.SemaphoreType.DMA((2,)),
            pltpu.SemaphoreType.DMA((2,)),
        ],
    )
    def gather_kernel(idx_hbm, table_hbm, out_hbm, idx_v, rows_v, gsem, osem):
        wid = lax.axis_index("s") * NC + lax.axis_index("c")
        ir_base = wid * ir_per_w

        def fire(c, b):
            ir0 = ir_base + c * CPW_IR
            pltpu.sync_copy(idx_hbm.at[pl.ds(ir0, CPW_IR)], idx_v.at[b])
            for j in range(CPW_IR):
                pltpu.async_copy(
                    table_hbm.at[idx_v.at[b, j]],
                    rows_v.at[b, pl.ds(j * IW, IW)],
                    gsem.at[b])

        def retire(c, b):
            ir0 = ir_base + c * CPW_IR
            row0 = ir0 * IW
            pltpu.make_async_copy(
                out_hbm.at[pl.ds(row0, RPC)], rows_v.at[b], gsem.at[b]).wait()
            pltpu.async_copy(
                rows_v.at[b], out_hbm.at[pl.ds(row0, RPC)], osem.at[b])

        def drain_out(b):
            pltpu.make_async_copy(
                out_hbm.at[pl.ds(ir_base * IW, RPC)], rows_v.at[b],
                osem.at[b]).wait()

        fire(0, 0)
        fire(1, 1)
        retire(0, 0)
        drain_out(0)
        fire(2, 0)
        retire(1, 1)

        def body(g, carry):
            c0 = 2 * g
            drain_out(1)
            fire(c0 + 1, 1)
            retire(c0, 0)
            drain_out(0)
            fire(c0 + 2, 0)
            retire(c0 + 1, 1)
            return carry

        lax.fori_loop(1, n_chunks // 2 - 1, body, 0)

        c0 = n_chunks - 2
        drain_out(1)
        fire(c0 + 1, 1)
        retire(c0, 0)
        retire(c0 + 1, 1)
        drain_out(0)
        drain_out(1)

    return gather_kernel


def kernel(token_ids, w):
    B = token_ids.shape[0] * token_ids.shape[1]
    flat = token_ids.reshape(B // IW, IW).astype(jnp.int32)
    out = _build(B, w.shape[1])(flat, w)
    return out.reshape(*token_ids.shape, w.shape[1])
